# initial kernel scaffold (unmeasured)
import jax
import jax.numpy as jnp
from jax import lax
from jax.experimental import pallas as pl
from jax.experimental.pallas import tpu as pltpu

N_DEV = 4
MB = 1024
CB = 1024


def kernel(x, w_mat):
    M, K_sh = x.shape
    _, N = w_mat.shape
    half = N // 2
    n_pass = half // CB

    def body(x_hbm, w_hbm, out_hbm,
             comm_fw, comm_bw, wt_fw, wt_bw, xc_fw, xc_bw, tmp_fw, tmp_bw,
             ss_fw, rs_fw, ss_bw, rs_bw, load_sems, store_sems):
        my = lax.axis_index("i")
        right = lax.rem(my + 1, N_DEV)
        left = lax.rem(my + N_DEV - 1, N_DEV)

        bar = pltpu.get_barrier_semaphore()
        for nbr in (left, right):
            pl.semaphore_signal(bar, inc=1, device_id=(nbr,),
                                device_id_type=pl.DeviceIdType.MESH)
        pl.semaphore_wait(bar, 2)

        def remote(comm, snd, rcv, ss, rs, dev):
            return pltpu.make_async_remote_copy(
                src_ref=comm.at[snd], dst_ref=comm.at[rcv],
                send_sem=ss.at[snd], recv_sem=rs.at[rcv],
                device_id=(dev,), device_id_type=pl.DeviceIdType.MESH)

        def load_x(chunk, dst, sem_slot):
            cp = pltpu.make_async_copy(
                x_hbm.at[pl.ds(chunk * MB, MB), :], dst, load_sems.at[sem_slot])
            cp.start()
            return cp

        def store_out(src, chunk, col_off, sem_slot):
            cp = pltpu.make_async_copy(
                src, out_hbm.at[pl.ds(chunk * MB, MB), pl.ds(col_off, CB)],
                store_sems.at[sem_slot])
            cp.start()
            return cp

        for p in range(n_pass):
            off_fw = p * CB
            off_bw = half + p * CB

            cw_f = pltpu.make_async_copy(
                w_hbm.at[:, pl.ds(off_fw, CB)], wt_fw, load_sems.at[0])
            cw_b = pltpu.make_async_copy(
                w_hbm.at[:, pl.ds(off_bw, CB)], wt_bw, load_sems.at[1])
            cw_f.start()
            cw_b.start()
            cx = load_x(my, xc_fw, 2)
            cw_f.wait()
            cw_b.wait()
            cx.wait()
            comm_fw[0, :, :] = jnp.dot(xc_fw[:, :], wt_fw[:, :],
                                       preferred_element_type=jnp.float32)
            comm_bw[0, :, :] = jnp.dot(xc_fw[:, :], wt_bw[:, :],
                                       preferred_element_type=jnp.float32)

            for s in range(N_DEV - 1):
                snd, rcv = s % 2, (s + 1) % 2
                r_fw = remote(comm_fw, snd, rcv, ss_fw, rs_fw, right)
                r_bw = remote(comm_bw, snd, rcv, ss_bw, rs_bw, left)
                r_fw.start()
                r_bw.start()
                c_fw = lax.rem(my - s - 1 + 2 * N_DEV, N_DEV)
                c_bw = lax.rem(my + s + 1, N_DEV)
                cxf = load_x(c_fw, xc_fw, 2)
                cxb = load_x(c_bw, xc_bw, 3)
                cxf.wait()
                tmp_fw[:, :] = jnp.dot(xc_fw[:, :], wt_fw[:, :],
                                       preferred_element_type=jnp.float32)
                cxb.wait()
                tmp_bw[:, :] = jnp.dot(xc_bw[:, :], wt_bw[:, :],
                                       preferred_element_type=jnp.float32)
                r_fw.wait()
                comm_fw[rcv, :, :] = comm_fw[rcv, :, :] + tmp_fw[:, :]
                r_bw.wait()
                comm_bw[rcv, :, :] = comm_bw[rcv, :, :] + tmp_bw[:, :]

            own_fw = lax.rem(my + 1, N_DEV)
            own_bw = lax.rem(my + N_DEV - 1, N_DEV)
            st_f = store_out(comm_fw.at[1], own_fw, off_fw, 0)
            st_b = store_out(comm_bw.at[1], own_bw, off_bw, 1)
            st_f.wait()
            st_b.wait()

            for t in range(N_DEV - 1):
                step = N_DEV - 1 + t
                snd, rcv = step % 2, (step + 1) % 2
                r_fw = remote(comm_fw, snd, rcv, ss_fw, rs_fw, right)
                r_bw = remote(comm_bw, snd, rcv, ss_bw, rs_bw, left)
                r_fw.start()
                r_bw.start()
                r_fw.wait()
                r_bw.wait()
                g_fw = lax.rem(my - t + 2 * N_DEV, N_DEV)
                g_bw = lax.rem(my + t, N_DEV)
                st_f = store_out(comm_fw.at[rcv], g_fw, off_fw, 0)
                st_b = store_out(comm_bw.at[rcv], g_bw, off_bw, 1)
                st_f.wait()
                st_b.wait()

    return pl.pallas_call(
        body,
        out_shape=jax.ShapeDtypeStruct((M, N), jnp.float32),
        in_specs=[
            pl.BlockSpec(memory_space=pltpu.ANY),
            pl.BlockSpec(memory_space=pltpu.ANY),
        ],
        out_specs=pl.BlockSpec(memory_space=pltpu.ANY),
        scratch_shapes=[
            pltpu.VMEM((2, MB, CB), jnp.float32),
            pltpu.VMEM((2, MB, CB), jnp.float32),
            pltpu.VMEM((K_sh, CB), jnp.float32),
            pltpu.VMEM((K_sh, CB), jnp.float32),
            pltpu.VMEM((MB, K_sh), jnp.float32),
            pltpu.VMEM((MB, K_sh), jnp.float32),
            pltpu.VMEM((MB, CB), jnp.float32),
            pltpu.VMEM((MB, CB), jnp.float32),
            pltpu.SemaphoreType.DMA((2,)),
            pltpu.SemaphoreType.DMA((2,)),
            pltpu.SemaphoreType.DMA((2,)),
            pltpu.SemaphoreType.DMA((2,)),
            pltpu.SemaphoreType.DMA((4,)),
            pltpu.SemaphoreType.DMA((2,)),
        ],
        compiler_params=pltpu.CompilerParams(collective_id=0),
    )(x, w_mat)


# baseline (device time: 1230140 ns/iter reference)
import jax
import jax.numpy as jnp
from jax import lax
from jax.experimental import pallas as pl
from jax.experimental.pallas import tpu as pltpu

N_DEV = 4
MB = 1024
CB = 1024


def kernel(x, w_mat):
    M, K_sh = x.shape
    _, N = w_mat.shape
    half = N // 2
    n_pass = half // CB

    def body(x_hbm, w_hbm, out_hbm,
             comm_fw, comm_bw, wt_fw, wt_bw, xc, xmy,
             init_fw, init_bw, tmp_fw, tmp_bw,
             ss_fw, rs_fw, ss_bw, rs_bw, load_sems, store_sems):
        my = lax.axis_index("i")
        right = lax.rem(my + 1, N_DEV)
        left = lax.rem(my + N_DEV - 1, N_DEV)

        bar = pltpu.get_barrier_semaphore()
        for nbr in (left, right):
            pl.semaphore_signal(bar, inc=1, device_id=(nbr,),
                                device_id_type=pl.DeviceIdType.MESH)
        pl.semaphore_wait(bar, 2)

        def remote(src, comm, rcv, ss, snd, rs, dev):
            return pltpu.make_async_remote_copy(
                src_ref=src, dst_ref=comm.at[rcv],
                send_sem=ss.at[snd], recv_sem=rs.at[rcv],
                device_id=(dev,), device_id_type=pl.DeviceIdType.MESH)

        def load_x(chunk, dst, sem_slot):
            cp = pltpu.make_async_copy(
                x_hbm.at[pl.ds(chunk * MB, MB), :], dst, load_sems.at[sem_slot])
            cp.start()
            return cp

        def load_w(col_off, dst, sem_slot):
            cp = pltpu.make_async_copy(
                w_hbm.at[:, pl.ds(col_off, CB)], dst, load_sems.at[sem_slot])
            cp.start()
            return cp

        pending = {}

        def store_out(src, chunk, col_off, dir_idx, k):
            key = (dir_idx, k)
            if key in pending:
                pending.pop(key).wait()
            cp = pltpu.make_async_copy(
                src, out_hbm.at[pl.ds(chunk * MB, MB), pl.ds(col_off, CB)],
                store_sems.at[dir_idx, k])
            cp.start()
            pending[key] = cp

        cwf = load_w(0, wt_fw.at[0], 0)
        cwb = load_w(half, wt_bw.at[0], 1)
        cx = load_x(my, xmy, 2)
        cwf.wait()
        cwb.wait()
        cx.wait()
        init_fw[:, :] = jnp.dot(xmy[:, :], wt_fw[0, :, :],
                                preferred_element_type=jnp.float32)
        init_bw[:, :] = jnp.dot(xmy[:, :], wt_bw[0, :, :],
                                preferred_element_type=jnp.float32)

        for p in range(n_pass):
            wi = p % 2
            off_fw = p * CB
            off_bw = half + p * CB

            for s in range(N_DEV - 1):
                snd, rcv = s % 2, (s + 1) % 2
                src_fw = init_fw if s == 0 else comm_fw.at[snd]
                src_bw = init_bw if s == 0 else comm_bw.at[snd]
                r_fw = remote(src_fw, comm_fw, rcv, ss_fw, snd, rs_fw, right)
                r_bw = remote(src_bw, comm_bw, rcv, ss_bw, snd, rs_bw, left)
                r_fw.start()
                r_bw.start()
                c_fw = lax.rem(my - s - 1 + 2 * N_DEV, N_DEV)
                c_bw = lax.rem(my + s + 1, N_DEV)
                cxf = load_x(c_fw, xc, 2)
                cxf.wait()
                tmp_fw[:, :] = jnp.dot(xc[:, :], wt_fw[wi, :, :],
                                       preferred_element_type=jnp.float32)
                cxb = load_x(c_bw, xc, 2)
                cxb.wait()
                tmp_bw[:, :] = jnp.dot(xc[:, :], wt_bw[wi, :, :],
                                       preferred_element_type=jnp.float32)
                r_fw.wait()
                comm_fw[rcv, :, :] = comm_fw[rcv, :, :] + tmp_fw[:, :]
                r_bw.wait()
                comm_bw[rcv, :, :] = comm_bw[rcv, :, :] + tmp_bw[:, :]

            own_fw = lax.rem(my + 1, N_DEV)
            own_bw = lax.rem(my + N_DEV - 1, N_DEV)

            for t in range(N_DEV - 1):
                step = N_DEV - 1 + t
                snd, rcv = step % 2, (step + 1) % 2
                r_fw = remote(comm_fw.at[snd], comm_fw, rcv, ss_fw, snd,
                              rs_fw, right)
                r_bw = remote(comm_bw.at[snd], comm_bw, rcv, ss_bw, snd,
                              rs_bw, left)
                r_fw.start()
                r_bw.start()
                if t == 0:
                    store_out(comm_fw.at[1], own_fw, off_fw, 0, 0)
                    store_out(comm_bw.at[1], own_bw, off_bw, 1, 0)
                    if p + 1 < n_pass:
                        cwf = load_w(off_fw + CB, wt_fw.at[1 - wi], 0)
                        cwb = load_w(off_bw + CB, wt_bw.at[1 - wi], 1)
                else:
                    g_fw = lax.rem(my - (t - 1) + 2 * N_DEV, N_DEV)
                    g_bw = lax.rem(my + (t - 1), N_DEV)
                    store_out(comm_fw.at[snd], g_fw, off_fw, 0, t)
                    store_out(comm_bw.at[snd], g_bw, off_bw, 1, t)
                if t == 1 and p + 1 < n_pass:
                    cwf.wait()
                    cwb.wait()
                    init_fw[:, :] = jnp.dot(xmy[:, :], wt_fw[1 - wi, :, :],
                                            preferred_element_type=jnp.float32)
                    init_bw[:, :] = jnp.dot(xmy[:, :], wt_bw[1 - wi, :, :],
                                            preferred_element_type=jnp.float32)
                r_fw.wait()
                r_bw.wait()

            store_out(comm_fw.at[0], lax.rem(my - 2 + 2 * N_DEV, N_DEV),
                      off_fw, 0, 3)
            store_out(comm_bw.at[0], lax.rem(my + 2, N_DEV), off_bw, 1, 3)

        for cp in pending.values():
            cp.wait()

    return pl.pallas_call(
        body,
        out_shape=jax.ShapeDtypeStruct((M, N), jnp.float32),
        in_specs=[
            pl.BlockSpec(memory_space=pl.ANY),
            pl.BlockSpec(memory_space=pl.ANY),
        ],
        out_specs=pl.BlockSpec(memory_space=pl.ANY),
        scratch_shapes=[
            pltpu.VMEM((2, MB, CB), jnp.float32),
            pltpu.VMEM((2, MB, CB), jnp.float32),
            pltpu.VMEM((2, K_sh, CB), jnp.float32),
            pltpu.VMEM((2, K_sh, CB), jnp.float32),
            pltpu.VMEM((MB, K_sh), jnp.float32),
            pltpu.VMEM((MB, K_sh), jnp.float32),
            pltpu.VMEM((MB, CB), jnp.float32),
            pltpu.VMEM((MB, CB), jnp.float32),
            pltpu.VMEM((MB, CB), jnp.float32),
            pltpu.VMEM((MB, CB), jnp.float32),
            pltpu.SemaphoreType.DMA((2,)),
            pltpu.SemaphoreType.DMA((2,)),
            pltpu.SemaphoreType.DMA((2,)),
            pltpu.SemaphoreType.DMA((2,)),
            pltpu.SemaphoreType.DMA((4,)),
            pltpu.SemaphoreType.DMA((2, 4)),
        ],
        compiler_params=pltpu.CompilerParams(
            collective_id=0, vmem_limit_bytes=60 * 1024 * 1024),
    )(x, w_mat)
